# baseline (speedup 1.0000x reference)
import jax
import jax.numpy as jnp
from jax import lax
from jax.experimental import pallas as pl
from jax.experimental.pallas import tpu as pltpu
from jax.experimental.pallas import tpu_sc as plsc

N = 10000
NP = 10240
E = 160000
D = 256
H = 128
NS = 16
CHUNK = 80
NCH = E // NS // CHUNK
SBC = 5
SB = NCH // SBC
NPT = NP // NS
FCH = 32
NF = NPT // FCH
ND = NP // 8
NDT = ND // NS


def _sage_body(x_ref, src_ref, dst_ref, pat_ref, za_ref, out_ref,
               src_v, dst_v, d3_v, d7_v, rows_v, facc_v, fdeg_v, acc, accd):
    c = lax.axis_index("c")
    s = lax.axis_index("s")

    pltpu.sync_copy(za_ref, facc_v)

    def zero_blk(i, _):
        pltpu.sync_copy(facc_v, acc.at[pl.ds(s * NPT + i * FCH, FCH)])
        return 0

    lax.fori_loop(0, NF, zero_blk, 0)

    def zero_deg(i, _):
        pltpu.sync_copy(facc_v, accd.at[pl.ds(s * NDT + i * FCH, FCH)])
        return 0

    lax.fori_loop(0, NDT // FCH, zero_deg, 0)
    pltpu.sync_copy(facc_v.at[pl.ds(0, NDT % FCH)],
                    accd.at[pl.ds(s * NDT + (NDT // FCH) * FCH, NDT % FCH)])

    off = jnp.full((16,), c * NP, jnp.int32)

    plsc.subcore_barrier()

    def block(b, _):
        pltpu.sync_copy(src_ref.at[s * SB + b], src_v)
        pltpu.sync_copy(dst_ref.at[s * SB + b], dst_v)

        def adj_row(i, _):
            def adj_lane(k, _):
                sl = pl.ds(k * 16, 16)
                src_v[i, sl] = src_v[i, sl] + off
                dv = dst_v[i, sl]
                d3_v[i, sl] = lax.shift_right_logical(dv, 3)
                d7_v[i, sl] = lax.bitwise_and(dv, 7)
                return 0
            return lax.fori_loop(0, CHUNK // 16, adj_lane, 0)

        lax.fori_loop(0, SBC, adj_row, 0)

        def chunk(j, _):
            pltpu.sync_copy(x_ref.at[src_v.at[j]], rows_v)
            pltpu.sync_copy(rows_v, acc.at[dst_v.at[j]], add=True)
            pltpu.sync_copy(pat_ref.at[d7_v.at[j]], rows_v)
            pltpu.sync_copy(rows_v, accd.at[d3_v.at[j]], add=True)
            return 0

        lax.fori_loop(0, SBC, chunk, 0)
        return 0

    lax.fori_loop(0, SB, block, 0)

    plsc.subcore_barrier()

    def fin(q, _):
        r0 = s * NPT + q * FCH
        pltpu.sync_copy(acc.at[pl.ds(r0, FCH)], facc_v)
        @pl.when(lax.rem(q, 2) == 0)
        def _():
            pltpu.sync_copy(accd.at[pl.ds(s * NDT + (q // 2) * 8, 8)], fdeg_v)

        n_base = lax.rem(q, 2) * FCH

        def row(r, _):
            n_off = n_base + r
            dsp = fdeg_v[n_off // 8, pl.ds(lax.rem(n_off, 8) * 16, 16)]
            splat = 1.0 / jnp.maximum(dsp, 1.0)

            def col(k, _):
                sl = pl.ds(k * 16, 16)
                facc_v[r, sl] = facc_v[r, sl] * splat
                return 0
            return lax.fori_loop(0, H // 16, col, 0)

        lax.fori_loop(0, FCH, row, 0)
        pltpu.sync_copy(facc_v, out_ref.at[pl.ds(c * NP + r0, FCH)])
        return 0

    lax.fori_loop(0, NF, fin, 0)


@jax.jit
def _sage(x_cat, src3d, dst3d, pat, za):
    mesh = plsc.VectorSubcoreMesh(core_axis_name="c", subcore_axis_name="s")
    return pl.kernel(
        _sage_body,
        out_type=jax.ShapeDtypeStruct((2 * NP, H), jnp.float32),
        mesh=mesh,
        scratch_types=[
            pltpu.VMEM((SBC, CHUNK), jnp.int32),
            pltpu.VMEM((SBC, CHUNK), jnp.int32),
            pltpu.VMEM((SBC, CHUNK), jnp.int32),
            pltpu.VMEM((SBC, CHUNK), jnp.int32),
            pltpu.VMEM((CHUNK, H), jnp.float32),
            pltpu.VMEM((FCH, H), jnp.float32),
            pltpu.VMEM((8, H), jnp.float32),
            pltpu.VMEM_SHARED((NP, H), jnp.float32),
            pltpu.VMEM_SHARED((ND, H), jnp.float32),
        ],
    )(x_cat, src3d, dst3d, pat, za)


def kernel(x_feat, edge_index):
    xp = jnp.pad(x_feat, ((0, NP - N), (0, 0)))
    x_cat = jnp.concatenate([xp[:, :H], xp[:, H:]], axis=0)
    src3d = edge_index[0].reshape(NS * SB, SBC, CHUNK)
    dst3d = edge_index[1].reshape(NS * SB, SBC, CHUNK)
    pat = (jnp.arange(H, dtype=jnp.int32)[None, :] // 16
           == jnp.arange(8, dtype=jnp.int32)[:, None]).astype(jnp.float32)
    za = jnp.zeros((FCH, H), jnp.float32)
    out = _sage(x_cat, src3d, dst3d, pat, za)
    mp_emb = out.reshape(2, NP, H)[:, :N].transpose(1, 0, 2).reshape(N, D)
    return (mp_emb, x_feat)
